# Initial kernel scaffold; baseline (speedup 1.0000x reference)
import jax
import jax.numpy as jnp
from jax.experimental import pallas as pl


def kernel(x, W, b):
    raise NotImplementedError("write your pallas kernel here")


# speedup vs baseline: 14.7281x; 14.7281x over previous
import jax
import jax.numpy as jnp
from jax.experimental import pallas as pl

K = 16
R = 256


def _tile_kernel(x_ref, w_ref, b_ref, o_ref):
    t = pl.program_id(1)
    xb = x_ref[0]
    n = xb.shape[0]
    xt = x_ref[0, pl.ds(t * R, R), :]

    sq_all = jnp.sum(xb * xb, axis=1)[None, :]
    sq_t = jnp.sum(xt * xt, axis=1)[:, None]
    g = jax.lax.dot_general(xt, xb, (((1,), (1,)), ((), ())),
                            preferred_element_type=jnp.float32)
    d = sq_t + sq_all - 2.0 * g

    w1 = w_ref[0:32, :]
    w2 = w_ref[32:64, :]
    y = jnp.dot(xb, w1, preferred_element_type=jnp.float32)
    z = jnp.dot(xt, w2 - w1, preferred_element_type=jnp.float32)
    z = z + b_ref[0][None, :]

    iota = jax.lax.broadcasted_iota(jnp.int32, (R, n), 1)
    acc = jnp.full((R, 64), -jnp.inf, dtype=jnp.float32)
    for _ in range(K):
        m = jnp.min(d, axis=1, keepdims=True)
        ind = jnp.min(jnp.where(d == m, iota, n), axis=1, keepdims=True)
        sel = iota == ind
        oh = sel.astype(jnp.float32)
        acc = jnp.maximum(
            acc,
            jax.lax.dot_general(oh, y, (((1,), (0,)), ((), ())),
                                preferred_element_type=jnp.float32))
        d = jnp.where(sel, jnp.inf, d)

    o_ref[0] = acc + z


def kernel(x, W, b):
    B, n, c = x.shape
    co = W.shape[1]
    b2 = b.reshape(1, co)
    return pl.pallas_call(
        _tile_kernel,
        grid=(B, n // R),
        in_specs=[
            pl.BlockSpec((1, n, c), lambda bi, ti: (bi, 0, 0)),
            pl.BlockSpec(W.shape, lambda bi, ti: (0, 0)),
            pl.BlockSpec((1, co), lambda bi, ti: (0, 0)),
        ],
        out_specs=pl.BlockSpec((1, R, co), lambda bi, ti: (bi, ti, 0)),
        out_shape=jax.ShapeDtypeStruct((B, n, co), jnp.float32),
    )(x, W, b2)
